# baseline (speedup 1.0000x reference)
import jax
import jax.numpy as jnp
from jax.experimental import pallas as pl

N_TOKENS = 16384
IN_FEATURES = 4096
N_EXPERTS = 64
TOP_K = 8
BM = 1024


def _gate_kernel(x_ref, w_ref, w_out_ref, i_out_ref):
    lt = jax.lax.dot_general(
        w_ref[...],
        x_ref[...],
        (((1,), (1,)), ((), ())),
        preferred_element_type=jnp.float32,
    )
    m = jnp.max(lt, axis=0, keepdims=True)
    e = jnp.exp(lt - m)
    z = jnp.sum(e, axis=0, keepdims=True)
    s = e / z

    iota = jax.lax.broadcasted_iota(jnp.int32, (N_EXPERTS, BM), 0)
    for j in range(TOP_K):
        cur = jnp.max(s, axis=0, keepdims=True)
        hit = s == cur
        idx = jnp.min(jnp.where(hit, iota, N_EXPERTS), axis=0, keepdims=True)
        w_out_ref[j : j + 1, :] = cur
        i_out_ref[j : j + 1, :] = idx
        s = jnp.where(iota == idx, -1.0, s)


def kernel(x, W):
    grid = (N_TOKENS // BM,)
    weights_t, indices_t = pl.pallas_call(
        _gate_kernel,
        grid=grid,
        in_specs=[
            pl.BlockSpec((BM, IN_FEATURES), lambda i: (i, 0)),
            pl.BlockSpec((N_EXPERTS, IN_FEATURES), lambda i: (0, 0)),
        ],
        out_specs=[
            pl.BlockSpec((TOP_K, BM), lambda i: (0, i)),
            pl.BlockSpec((TOP_K, BM), lambda i: (0, i)),
        ],
        out_shape=[
            jax.ShapeDtypeStruct((TOP_K, N_TOKENS), jnp.float32),
            jax.ShapeDtypeStruct((TOP_K, N_TOKENS), jnp.int32),
        ],
    )(x, W)
    return weights_t.T, indices_t.T
